# speedup vs baseline: 5.0576x; 5.0576x over previous
import jax
import jax.numpy as jnp
from jax.experimental import pallas as pl

_BS = 512


def _copy_kernel(pos_ref, out_ref):
    blk = pos_ref[...]
    out_ref[...] = jnp.broadcast_to(blk[None, :, :], out_ref.shape)


def kernel(x, pos_emb):
    B, S = x.shape
    N, H = pos_emb.shape
    grid = (S // _BS,)
    out = pl.pallas_call(
        _copy_kernel,
        grid=grid,
        in_specs=[pl.BlockSpec((_BS, H), lambda j: (j, 0))],
        out_specs=pl.BlockSpec((B, _BS, H), lambda j: (0, j, 0)),
        out_shape=jax.ShapeDtypeStruct((B, S, H), pos_emb.dtype),
    )(pos_emb)
    return out
